# baseline (speedup 1.0000x reference)
import functools

import jax
import jax.numpy as jnp
from jax import lax
from jax.experimental import pallas as pl
from jax.experimental.pallas import tpu as pltpu
from jax.experimental.pallas import tpu_sc as plsc

N = 10000
E = 320000
D_EDGE = 16
L = 128

BN = 2000
BE = 4000

NC = 2
NS = 16
NW = NC * NS
TPE = E // NW
H = E // 2
C = 40
TPT = H // NS
NCHUNK = TPT // C
NDC = N // C


def _ln(y, g, b):
    m = jnp.mean(y, axis=-1, keepdims=True)
    v = jnp.mean((y - m) ** 2, axis=-1, keepdims=True)
    return (y - m) / jnp.sqrt(v + 1e-5) * g + b


def _enc_node_body(x_ref, w0, b0, w1, b1, g, bl, wa, wb, xo, po, qo):
    h = jnp.maximum(x_ref[...] @ w0[...] + b0[...], 0.0)
    xn = _ln(h @ w1[...] + b1[...], g[...], bl[...])
    xo[...] = xn
    po[...] = xn @ wa[...]
    qo[...] = xn @ wb[...]


def _enc_node_pq(node_x, w0, b0, w1, b1, g, bl, wa, wb):
    full = pl.BlockSpec((L, L), lambda i: (0, 0))
    vec = pl.BlockSpec((1, L), lambda i: (0, 0))
    blk = pl.BlockSpec((BN, L), lambda i: (i, 0))
    return pl.pallas_call(
        _enc_node_body,
        grid=(N // BN,),
        in_specs=[blk, full, vec, full, vec, vec, vec, full, full],
        out_specs=[blk, blk, blk],
        out_shape=[jax.ShapeDtypeStruct((N, L), jnp.float32)] * 3,
    )(node_x, w0, b0, w1, b1, g, bl, wa, wb)


def _enc_edge_body(a_ref, w0, b0, w1, b1, g, bl, eo):
    h = jnp.maximum(a_ref[...] @ w0[...] + b0[...], 0.0)
    eo[...] = _ln(h @ w1[...] + b1[...], g[...], bl[...])


def _enc_edge(edge_attr, off, w0, b0, w1, b1, g, bl):
    vec = pl.BlockSpec((1, L), lambda i: (0, 0))
    return pl.pallas_call(
        _enc_edge_body,
        grid=(H // BE,),
        in_specs=[pl.BlockSpec((BE, D_EDGE), lambda i: (i + off, 0)),
                  pl.BlockSpec((D_EDGE, L), lambda i: (0, 0)),
                  vec,
                  pl.BlockSpec((L, L), lambda i: (0, 0)),
                  vec, vec, vec],
        out_specs=pl.BlockSpec((BE, L), lambda i: (i, 0)),
        out_shape=jax.ShapeDtypeStruct((H, L), jnp.float32),
    )(edge_attr, w0, b0, w1, b1, g, bl)


def _msg0_body(a_ref, b_ref, ea_ref, ew0, eb0, ew1, eb1, eg, ebl,
               wc, b0, w1, b1, g, bl, eo):
    he = jnp.maximum(ea_ref[...] @ ew0[...] + eb0[...], 0.0)
    e = _ln(he @ ew1[...] + eb1[...], eg[...], ebl[...])
    gsum = a_ref[...].astype(jnp.float32) + b_ref[...].astype(jnp.float32)
    pre = gsum + e @ wc[...] + b0[...]
    h = jnp.maximum(pre, 0.0)
    msg = _ln(h @ w1[...] + b1[...], g[...], bl[...])
    eo[...] = e + msg


def _msg0_update(a, b, edge_attr, off, ew0, eb0, ew1, eb1, eg, ebl,
                 wc, b0, w1, b1, g, bl):
    full = pl.BlockSpec((L, L), lambda i: (0, 0))
    vec = pl.BlockSpec((1, L), lambda i: (0, 0))
    blk = pl.BlockSpec((BE, L), lambda i: (i, 0))
    return pl.pallas_call(
        _msg0_body,
        grid=(H // BE,),
        in_specs=[blk, blk,
                  pl.BlockSpec((BE, D_EDGE), lambda i: (i + off, 0)),
                  pl.BlockSpec((D_EDGE, L), lambda i: (0, 0)),
                  vec, full, vec, vec, vec,
                  full, vec, full, vec, vec, vec],
        out_specs=pl.BlockSpec((BE, L), lambda i: (i, 0)),
        out_shape=jax.ShapeDtypeStruct((H, L), jnp.float32),
    )(a, b, edge_attr, ew0, eb0, ew1, eb1, eg, ebl,
      wc, b0, w1, b1, g, bl)


def _msg_body(a_ref, b_ref, e_ref, wc, b0, w1, b1, g, bl, eo):
    gsum = a_ref[...].astype(jnp.float32) + b_ref[...].astype(jnp.float32)
    pre = gsum + e_ref[...] @ wc[...] + b0[...]
    h = jnp.maximum(pre, 0.0)
    msg = _ln(h @ w1[...] + b1[...], g[...], bl[...])
    eo[...] = e_ref[...] + msg


def _msg_update(a, b, e, wc, b0, w1, b1, g, bl):
    full = pl.BlockSpec((L, L), lambda i: (0, 0))
    vec = pl.BlockSpec((1, L), lambda i: (0, 0))
    blk = pl.BlockSpec((BE, L), lambda i: (i, 0))
    return pl.pallas_call(
        _msg_body,
        grid=(H // BE,),
        in_specs=[blk, blk, blk, full, vec, full, vec, vec, vec],
        out_specs=blk,
        out_shape=jax.ShapeDtypeStruct((H, L), jnp.float32),
    )(a, b, e, wc, b0, w1, b1, g, bl)


def _upd_pq_body(x_ref, agg_ref, wx, wg, b0, w1, b1, g, bl, wa, wb,
                 xo, po, qo):
    agg = agg_ref[0] + agg_ref[1]
    pre = x_ref[...] @ wx[...] + agg @ wg[...] + b0[...]
    h = jnp.maximum(pre, 0.0)
    upd = _ln(h @ w1[...] + b1[...], g[...], bl[...])
    xn = x_ref[...] + upd
    xo[...] = xn
    po[...] = xn @ wa[...]
    qo[...] = xn @ wb[...]


def _upd_pq(x, aggp, wx, wg, b0, w1, b1, g, bl, wa, wb):
    full = pl.BlockSpec((L, L), lambda i: (0, 0))
    vec = pl.BlockSpec((1, L), lambda i: (0, 0))
    blk = pl.BlockSpec((BN, L), lambda i: (i, 0))
    ablk = pl.BlockSpec((2, BN, L), lambda i: (0, i, 0))
    return pl.pallas_call(
        _upd_pq_body,
        grid=(N // BN,),
        in_specs=[blk, ablk, full, full, vec, full, vec, vec, vec, full, full],
        out_specs=[blk, blk, blk],
        out_shape=[jax.ShapeDtypeStruct((N, L), jnp.float32)] * 3,
    )(x, aggp, wx, wg, b0, w1, b1, g, bl, wa, wb)


def _upd_dec_body(x_ref, agg_ref, wx, wg, b0, w1, b1, g, bl,
                  wd0, bd0, wd1, bd1, yo):
    agg = agg_ref[0] + agg_ref[1]
    pre = x_ref[...] @ wx[...] + agg @ wg[...] + b0[...]
    h = jnp.maximum(pre, 0.0)
    upd = _ln(h @ w1[...] + b1[...], g[...], bl[...])
    xn = x_ref[...] + upd
    hd = jnp.maximum(xn @ wd0[...] + bd0[...], 0.0)
    yo[...] = hd @ wd1[...] + bd1[...]


def _upd_dec(x, aggp, wx, wg, b0, w1, b1, g, bl, wd0, bd0, wd1, bd1):
    full = pl.BlockSpec((L, L), lambda i: (0, 0))
    vec = pl.BlockSpec((1, L), lambda i: (0, 0))
    blk = pl.BlockSpec((BN, L), lambda i: (i, 0))
    ablk = pl.BlockSpec((2, BN, L), lambda i: (0, i, 0))
    return pl.pallas_call(
        _upd_dec_body,
        grid=(N // BN,),
        in_specs=[blk, ablk, full, full, vec, full, vec, vec, vec,
                  full, vec,
                  pl.BlockSpec((L, 3), lambda i: (0, 0)),
                  pl.BlockSpec((1, 3), lambda i: (0, 0))],
        out_specs=pl.BlockSpec((BN, 3), lambda i: (i, 0)),
        out_shape=jax.ShapeDtypeStruct((N, 3), jnp.float32),
    )(x, aggp, wx, wg, b0, w1, b1, g, bl, wd0, bd0, wd1, bd1)


NBG = 5
DG = 2
CG = C
NCH_G = TPT // CG
NGRP_G = NCH_G // NBG
NSTAGE = N // CG


def _sc_gather2(p, q, dst, src, eo):
    mesh = plsc.VectorSubcoreMesh(core_axis_name="c", subcore_axis_name="s")

    @functools.partial(
        pl.kernel, mesh=mesh,
        out_type=[jax.ShapeDtypeStruct((H, L), jnp.float32)] * 2,
        scratch_types=(
            [pltpu.VMEM((TPT,), jnp.int32)]
            + [pltpu.VMEM((CG, L), jnp.float32)] * NBG
            + [pltpu.VMEM_SHARED((N, L), jnp.float32)]
            + [pltpu.SemaphoreType.DMA] * (2 * NBG)
        ),
    )
    def k(p_hbm, q_hbm, dst_hbm, src_hbm, a_hbm, b_hbm, idxall, *rest):
        rows = rest[:NBG]
        tbl_sh = rest[NBG]
        gsem = rest[NBG + 1:NBG + 1 + NBG]
        wsem = rest[NBG + 1 + NBG:]
        cid = lax.axis_index("c")
        sid = lax.axis_index("s")
        base = sid * TPT

        def run(tbl_hbm, idx_hbm, out_hbm):
            for st in range(-(-NSTAGE // NS)):
                ch = sid + NS * st

                @pl.when(ch < NSTAGE)
                def _():
                    pltpu.sync_copy(tbl_hbm.at[pl.ds(ch * CG, CG)], rows[0])
                    pltpu.sync_copy(rows[0], tbl_sh.at[pl.ds(ch * CG, CG)])

            pltpu.sync_copy(idx_hbm.at[pl.ds(eo + base, TPT)], idxall)
            plsc.subcore_barrier()

            def grp(g, carry):
                for b in range(NBG):
                    c = g * NBG + b

                    @pl.when(g > 0)
                    def _():
                        pltpu.make_async_copy(
                            rows[b], out_hbm.at[pl.ds(base + b * CG, CG)],
                            wsem[b]).wait()

                    pltpu.async_copy(
                        tbl_sh.at[idxall.at[pl.ds(c * CG, CG)]], rows[b],
                        gsem[b])

                    cd = c - DG
                    bd = (b - DG) % NBG

                    @pl.when(cd >= 0)
                    def _():
                        pltpu.make_async_copy(
                            tbl_sh.at[idxall.at[pl.ds(bd * CG, CG)]], rows[bd],
                            gsem[bd]).wait()
                        pltpu.async_copy(
                            rows[bd], out_hbm.at[pl.ds(base + cd * CG, CG)],
                            wsem[bd])
                return carry

            lax.fori_loop(0, NGRP_G, grp, 0)

            for bd in range(NBG - DG, NBG):
                cd = NCH_G - NBG + bd
                pltpu.make_async_copy(
                    tbl_sh.at[idxall.at[pl.ds(bd * CG, CG)]], rows[bd],
                    gsem[bd]).wait()
                pltpu.async_copy(
                    rows[bd], out_hbm.at[pl.ds(base + cd * CG, CG)], wsem[bd])
            for b in range(NBG):
                pltpu.make_async_copy(
                    rows[b], out_hbm.at[pl.ds(base + b * CG, CG)],
                    wsem[b]).wait()

        @pl.when(cid == 0)
        def _():
            run(p_hbm, dst_hbm, a_hbm)

        @pl.when(cid == 1)
        def _():
            run(q_hbm, src_hbm, b_hbm)

    return k(p, q, dst, src)


NBS = 5
DS = 2


def _sc_scatter(rows1, rows2, dst):
    mesh = plsc.VectorSubcoreMesh(core_axis_name="c", subcore_axis_name="s")

    @functools.partial(
        pl.kernel, mesh=mesh,
        out_type=jax.ShapeDtypeStruct((NC, N, L), jnp.float32),
        scratch_types=(
            [pltpu.VMEM((C,), jnp.int32)] * NBS
            + [pltpu.VMEM((C, L), jnp.float32)] * NBS
            + [pltpu.VMEM_SHARED((N, L), jnp.float32)]
            + [pltpu.SemaphoreType.DMA] * (3 * NBS)
        ),
    )
    def k(rows1_hbm, rows2_hbm, dst_hbm, out_hbm, *rest):
        di = rest[:NBS]
        rows = rest[NBS:2 * NBS]
        acc_sh = rest[2 * NBS]
        isem = rest[2 * NBS + 1:2 * NBS + 1 + NBS]
        rsem = rest[2 * NBS + 1 + NBS:2 * NBS + 1 + 2 * NBS]
        ssem = rest[2 * NBS + 1 + 2 * NBS:]
        cid = lax.axis_index("c")
        sid = lax.axis_index("s")
        base = sid * TPT

        def zrow(i, carry):
            def zcol(j, carry2):
                rows[0][i, pl.ds(j * 16, 16)] = jnp.zeros((16,), jnp.float32)
                return carry2
            return lax.fori_loop(0, L // 16, zcol, carry)
        lax.fori_loop(0, C, zrow, 0)

        for kk in range(-(-NDC // NS)):
            ch = sid + NS * kk

            @pl.when(ch < NDC)
            def _():
                pltpu.sync_copy(rows[0], acc_sh.at[pl.ds(ch * C, C)])

        plsc.subcore_barrier()

        def run(rows_hbm, eo):
            def grp(g, carry):
                for b in range(NBS):
                    c = g * NBS + b
                    off = base + c * C

                    @pl.when(g > 0)
                    def _():
                        pltpu.make_async_copy(rows[b], acc_sh.at[di[b]],
                                              ssem[b]).wait()

                    pltpu.async_copy(dst_hbm.at[pl.ds(eo + off, C)], di[b],
                                     isem[b])
                    pltpu.async_copy(rows_hbm.at[pl.ds(off, C)], rows[b],
                                     rsem[b])

                    cd = c - DS
                    bd = (b - DS) % NBS
                    offd = base + cd * C

                    @pl.when(cd >= 0)
                    def _():
                        pltpu.make_async_copy(dst_hbm.at[pl.ds(eo + offd, C)],
                                              di[bd], isem[bd]).wait()
                        pltpu.make_async_copy(rows_hbm.at[pl.ds(offd, C)],
                                              rows[bd], rsem[bd]).wait()
                        pltpu.async_copy(rows[bd], acc_sh.at[di[bd]], ssem[bd],
                                         add=True)
                return carry

            lax.fori_loop(0, NCHUNK // NBS, grp, 0)

            for bd in range(NBS - DS, NBS):
                offd = base + (NCHUNK - NBS + bd) * C
                pltpu.make_async_copy(dst_hbm.at[pl.ds(eo + offd, C)], di[bd],
                                      isem[bd]).wait()
                pltpu.make_async_copy(rows_hbm.at[pl.ds(offd, C)], rows[bd],
                                      rsem[bd]).wait()
                pltpu.async_copy(rows[bd], acc_sh.at[di[bd]], ssem[bd],
                                 add=True)
            for b in range(NBS):
                pltpu.make_async_copy(rows[b], acc_sh.at[di[b]],
                                      ssem[b]).wait()

        @pl.when(cid == 0)
        def _():
            run(rows1_hbm, 0)

        @pl.when(cid == 1)
        def _():
            run(rows2_hbm, H)

        plsc.subcore_barrier()

        for kk in range(-(-NDC // NS)):
            ch = sid + NS * kk

            @pl.when(ch < NDC)
            def _():
                pltpu.sync_copy(acc_sh.at[pl.ds(ch * C, C)], rows[0])
                pltpu.sync_copy(rows[0], out_hbm.at[cid, pl.ds(ch * C, C)])

    return k(rows1, rows2, dst)


def _vec(b):
    return b.reshape(1, -1)


def kernel(node_x, edge_attr, params, edge_index, edge_type):
    del edge_type
    src = edge_index[0]
    dst = edge_index[1]

    (en_w0, en_b0), (en_w1, en_b1) = params["enc_node"]["mlp"]
    en_g, en_bl = params["enc_node"]["ln"]
    (ee_w0, ee_b0), (ee_w1, ee_b1) = params["enc_edge"]["mlp"]
    ee_g, ee_bl = params["enc_edge"]["ln"]

    steps = []
    for st in params["proc"]:
        (mw0, mb0), (mw1, mb1) = st["msg"]["mlp"]
        mg, mbl = st["msg"]["ln"]
        (uw0, ub0), (uw1, ub1) = st["upd"]["mlp"]
        ug, ubl = st["upd"]["ln"]
        steps.append(dict(
            wa=mw0[:L], wb=mw0[L:2 * L], wc=mw0[2 * L:],
            mb0=_vec(mb0), mw1=mw1, mb1=_vec(mb1), mg=_vec(mg), mbl=_vec(mbl),
            wx=uw0[:L], wg=uw0[L:],
            ub0=_vec(ub0), uw1=uw1, ub1=_vec(ub1), ug=_vec(ug), ubl=_vec(ubl),
        ))
    (dw0, db0), (dw1, db1) = params["dec"]

    s0, s1 = steps
    x, p, q = _enc_node_pq(node_x, en_w0, _vec(en_b0), en_w1, _vec(en_b1),
                           _vec(en_g), _vec(en_bl), s0["wa"], s0["wb"])
    a1, b1 = _sc_gather2(p, q, dst, src, 0)
    a2, b2 = _sc_gather2(p, q, dst, src, H)
    enc_e = (ee_w0, _vec(ee_b0), ee_w1, _vec(ee_b1), _vec(ee_g), _vec(ee_bl))
    e1 = _msg0_update(a1, b1, edge_attr, 0,
                      *enc_e, s0["wc"], s0["mb0"], s0["mw1"], s0["mb1"],
                      s0["mg"], s0["mbl"])
    e2 = _msg0_update(a2, b2, edge_attr, H // BE,
                      *enc_e, s0["wc"], s0["mb0"], s0["mw1"], s0["mb1"],
                      s0["mg"], s0["mbl"])
    aggp = _sc_scatter(e1, e2, dst)
    x, p, q = _upd_pq(x, aggp, s0["wx"], s0["wg"], s0["ub0"], s0["uw1"],
                      s0["ub1"], s0["ug"], s0["ubl"], s1["wa"], s1["wb"])

    a1, b1 = _sc_gather2(p, q, dst, src, 0)
    a2, b2 = _sc_gather2(p, q, dst, src, H)
    e1 = _msg_update(a1, b1, e1, s1["wc"],
                     s1["mb0"], s1["mw1"], s1["mb1"], s1["mg"], s1["mbl"])
    e2 = _msg_update(a2, b2, e2, s1["wc"],
                     s1["mb0"], s1["mw1"], s1["mb1"], s1["mg"], s1["mbl"])
    aggp = _sc_scatter(e1, e2, dst)
    y = _upd_dec(x, aggp, s1["wx"], s1["wg"], s1["ub0"], s1["uw1"],
                 s1["ub1"], s1["ug"], s1["ubl"],
                 dw0, _vec(db0), dw1, _vec(db1))
    return y


# speedup vs baseline: 1.0138x; 1.0138x over previous
import functools

import jax
import jax.numpy as jnp
from jax import lax
from jax.experimental import pallas as pl
from jax.experimental.pallas import tpu as pltpu
from jax.experimental.pallas import tpu_sc as plsc

N = 10000
E = 320000
D_EDGE = 16
L = 128

BN = 2000
BE = 4000

NC = 2
NS = 16
NW = NC * NS
TPE = E // NW
H = E // 2
C = 40
TPT = H // NS
NCHUNK = TPT // C
NDC = N // C


def _ln(y, g, b):
    m = jnp.mean(y, axis=-1, keepdims=True)
    v = jnp.mean((y - m) ** 2, axis=-1, keepdims=True)
    return (y - m) / jnp.sqrt(v + 1e-5) * g + b


def _enc_node_body(x_ref, w0, b0, w1, b1, g, bl, wa, wb, xo, po, qo):
    h = jnp.maximum(x_ref[...] @ w0[...] + b0[...], 0.0)
    xn = _ln(h @ w1[...] + b1[...], g[...], bl[...])
    xo[...] = xn
    po[...] = xn @ wa[...]
    qo[...] = xn @ wb[...]


def _enc_node_pq(node_x, w0, b0, w1, b1, g, bl, wa, wb):
    full = pl.BlockSpec((L, L), lambda i: (0, 0))
    vec = pl.BlockSpec((1, L), lambda i: (0, 0))
    blk = pl.BlockSpec((BN, L), lambda i: (i, 0))
    return pl.pallas_call(
        _enc_node_body,
        grid=(N // BN,),
        in_specs=[blk, full, vec, full, vec, vec, vec, full, full],
        out_specs=[blk, blk, blk],
        out_shape=[jax.ShapeDtypeStruct((N, L), jnp.float32)] * 3,
    )(node_x, w0, b0, w1, b1, g, bl, wa, wb)


def _enc_edge_body(a_ref, w0, b0, w1, b1, g, bl, eo):
    h = jnp.maximum(a_ref[...] @ w0[...] + b0[...], 0.0)
    eo[...] = _ln(h @ w1[...] + b1[...], g[...], bl[...])


def _enc_edge(edge_attr, off, w0, b0, w1, b1, g, bl):
    vec = pl.BlockSpec((1, L), lambda i: (0, 0))
    return pl.pallas_call(
        _enc_edge_body,
        grid=(H // BE,),
        in_specs=[pl.BlockSpec((BE, D_EDGE), lambda i: (i + off, 0)),
                  pl.BlockSpec((D_EDGE, L), lambda i: (0, 0)),
                  vec,
                  pl.BlockSpec((L, L), lambda i: (0, 0)),
                  vec, vec, vec],
        out_specs=pl.BlockSpec((BE, L), lambda i: (i, 0)),
        out_shape=jax.ShapeDtypeStruct((H, L), jnp.float32),
    )(edge_attr, w0, b0, w1, b1, g, bl)


def _msg0_body(a_ref, b_ref, ea_ref, ew0, eb0, ew1, eb1, eg, ebl,
               wc, b0, w1, b1, g, bl, eo):
    he = jnp.maximum(ea_ref[...] @ ew0[...] + eb0[...], 0.0)
    e = _ln(he @ ew1[...] + eb1[...], eg[...], ebl[...])
    gsum = a_ref[...].astype(jnp.float32) + b_ref[...].astype(jnp.float32)
    pre = gsum + e @ wc[...] + b0[...]
    h = jnp.maximum(pre, 0.0)
    msg = _ln(h @ w1[...] + b1[...], g[...], bl[...])
    eo[...] = e + msg


def _msg0_update(a, b, edge_attr, off, ew0, eb0, ew1, eb1, eg, ebl,
                 wc, b0, w1, b1, g, bl):
    full = pl.BlockSpec((L, L), lambda i: (0, 0))
    vec = pl.BlockSpec((1, L), lambda i: (0, 0))
    blk = pl.BlockSpec((BE, L), lambda i: (i, 0))
    return pl.pallas_call(
        _msg0_body,
        grid=(H // BE,),
        in_specs=[blk, blk,
                  pl.BlockSpec((BE, D_EDGE), lambda i: (i + off, 0)),
                  pl.BlockSpec((D_EDGE, L), lambda i: (0, 0)),
                  vec, full, vec, vec, vec,
                  full, vec, full, vec, vec, vec],
        out_specs=pl.BlockSpec((BE, L), lambda i: (i, 0)),
        out_shape=jax.ShapeDtypeStruct((H, L), jnp.float32),
    )(a, b, edge_attr, ew0, eb0, ew1, eb1, eg, ebl,
      wc, b0, w1, b1, g, bl)


def _msg_body(a_ref, b_ref, e_ref, wc, b0, w1, b1, g, bl, eo):
    gsum = a_ref[...].astype(jnp.float32) + b_ref[...].astype(jnp.float32)
    pre = gsum + e_ref[...] @ wc[...] + b0[...]
    h = jnp.maximum(pre, 0.0)
    msg = _ln(h @ w1[...] + b1[...], g[...], bl[...])
    eo[...] = e_ref[...] + msg


def _msg_update(a, b, e, wc, b0, w1, b1, g, bl):
    full = pl.BlockSpec((L, L), lambda i: (0, 0))
    vec = pl.BlockSpec((1, L), lambda i: (0, 0))
    blk = pl.BlockSpec((BE, L), lambda i: (i, 0))
    return pl.pallas_call(
        _msg_body,
        grid=(H // BE,),
        in_specs=[blk, blk, blk, full, vec, full, vec, vec, vec],
        out_specs=blk,
        out_shape=jax.ShapeDtypeStruct((H, L), jnp.float32),
    )(a, b, e, wc, b0, w1, b1, g, bl)


def _upd_pq_body(x_ref, agg_ref, wx, wg, b0, w1, b1, g, bl, wa, wb,
                 xo, po, qo):
    agg = agg_ref[0] + agg_ref[1]
    pre = x_ref[...] @ wx[...] + agg @ wg[...] + b0[...]
    h = jnp.maximum(pre, 0.0)
    upd = _ln(h @ w1[...] + b1[...], g[...], bl[...])
    xn = x_ref[...] + upd
    xo[...] = xn
    po[...] = xn @ wa[...]
    qo[...] = xn @ wb[...]


def _upd_pq(x, aggp, wx, wg, b0, w1, b1, g, bl, wa, wb):
    full = pl.BlockSpec((L, L), lambda i: (0, 0))
    vec = pl.BlockSpec((1, L), lambda i: (0, 0))
    blk = pl.BlockSpec((BN, L), lambda i: (i, 0))
    ablk = pl.BlockSpec((2, BN, L), lambda i: (0, i, 0))
    return pl.pallas_call(
        _upd_pq_body,
        grid=(N // BN,),
        in_specs=[blk, ablk, full, full, vec, full, vec, vec, vec, full, full],
        out_specs=[blk, blk, blk],
        out_shape=[jax.ShapeDtypeStruct((N, L), jnp.float32)] * 3,
    )(x, aggp, wx, wg, b0, w1, b1, g, bl, wa, wb)


def _upd_dec_body(x_ref, agg_ref, wx, wg, b0, w1, b1, g, bl,
                  wd0, bd0, wd1, bd1, yo):
    agg = agg_ref[0] + agg_ref[1]
    pre = x_ref[...] @ wx[...] + agg @ wg[...] + b0[...]
    h = jnp.maximum(pre, 0.0)
    upd = _ln(h @ w1[...] + b1[...], g[...], bl[...])
    xn = x_ref[...] + upd
    hd = jnp.maximum(xn @ wd0[...] + bd0[...], 0.0)
    yo[...] = hd @ wd1[...] + bd1[...]


def _upd_dec(x, aggp, wx, wg, b0, w1, b1, g, bl, wd0, bd0, wd1, bd1):
    full = pl.BlockSpec((L, L), lambda i: (0, 0))
    vec = pl.BlockSpec((1, L), lambda i: (0, 0))
    blk = pl.BlockSpec((BN, L), lambda i: (i, 0))
    ablk = pl.BlockSpec((2, BN, L), lambda i: (0, i, 0))
    return pl.pallas_call(
        _upd_dec_body,
        grid=(N // BN,),
        in_specs=[blk, ablk, full, full, vec, full, vec, vec, vec,
                  full, vec,
                  pl.BlockSpec((L, 3), lambda i: (0, 0)),
                  pl.BlockSpec((1, 3), lambda i: (0, 0))],
        out_specs=pl.BlockSpec((BN, 3), lambda i: (i, 0)),
        out_shape=jax.ShapeDtypeStruct((N, 3), jnp.float32),
    )(x, aggp, wx, wg, b0, w1, b1, g, bl, wd0, bd0, wd1, bd1)


NBG = 5
DG = 1
CG = C
NCH_G = TPT // CG
NGRP_G = NCH_G // NBG
NSTAGE = N // CG


def _sc_gather2(p, q, dst, src, eo):
    mesh = plsc.VectorSubcoreMesh(core_axis_name="c", subcore_axis_name="s")

    @functools.partial(
        pl.kernel, mesh=mesh,
        out_type=[jax.ShapeDtypeStruct((H, L), jnp.float32)] * 2,
        scratch_types=(
            [pltpu.VMEM((TPT,), jnp.int32)]
            + [pltpu.VMEM((CG, L), jnp.float32)] * NBG
            + [pltpu.VMEM_SHARED((N, L), jnp.float32)]
            + [pltpu.SemaphoreType.DMA] * (2 * NBG)
        ),
    )
    def k(p_hbm, q_hbm, dst_hbm, src_hbm, a_hbm, b_hbm, idxall, *rest):
        rows = rest[:NBG]
        tbl_sh = rest[NBG]
        gsem = rest[NBG + 1:NBG + 1 + NBG]
        wsem = rest[NBG + 1 + NBG:]
        cid = lax.axis_index("c")
        sid = lax.axis_index("s")
        base = sid * TPT

        def run(tbl_hbm, idx_hbm, out_hbm):
            for st in range(-(-NSTAGE // NS)):
                ch = sid + NS * st

                @pl.when(ch < NSTAGE)
                def _():
                    pltpu.sync_copy(tbl_hbm.at[pl.ds(ch * CG, CG)], rows[0])
                    pltpu.sync_copy(rows[0], tbl_sh.at[pl.ds(ch * CG, CG)])

            pltpu.sync_copy(idx_hbm.at[pl.ds(eo + base, TPT)], idxall)
            plsc.subcore_barrier()

            def grp(g, carry):
                for b in range(NBG):
                    c = g * NBG + b

                    @pl.when(g > 0)
                    def _():
                        pltpu.make_async_copy(
                            rows[b], out_hbm.at[pl.ds(base + b * CG, CG)],
                            wsem[b]).wait()

                    pltpu.async_copy(
                        tbl_sh.at[idxall.at[pl.ds(c * CG, CG)]], rows[b],
                        gsem[b])

                    cd = c - DG
                    bd = (b - DG) % NBG

                    @pl.when(cd >= 0)
                    def _():
                        pltpu.make_async_copy(
                            tbl_sh.at[idxall.at[pl.ds(bd * CG, CG)]], rows[bd],
                            gsem[bd]).wait()
                        pltpu.async_copy(
                            rows[bd], out_hbm.at[pl.ds(base + cd * CG, CG)],
                            wsem[bd])
                return carry

            lax.fori_loop(0, NGRP_G, grp, 0)

            for bd in range(NBG - DG, NBG):
                cd = NCH_G - NBG + bd
                pltpu.make_async_copy(
                    tbl_sh.at[idxall.at[pl.ds(bd * CG, CG)]], rows[bd],
                    gsem[bd]).wait()
                pltpu.async_copy(
                    rows[bd], out_hbm.at[pl.ds(base + cd * CG, CG)], wsem[bd])
            for b in range(NBG):
                pltpu.make_async_copy(
                    rows[b], out_hbm.at[pl.ds(base + b * CG, CG)],
                    wsem[b]).wait()

        @pl.when(cid == 0)
        def _():
            run(p_hbm, dst_hbm, a_hbm)

        @pl.when(cid == 1)
        def _():
            run(q_hbm, src_hbm, b_hbm)

    return k(p, q, dst, src)


NBS = 5
DS = 3


def _sc_scatter(rows1, rows2, dst):
    mesh = plsc.VectorSubcoreMesh(core_axis_name="c", subcore_axis_name="s")

    @functools.partial(
        pl.kernel, mesh=mesh,
        out_type=jax.ShapeDtypeStruct((NC, N, L), jnp.float32),
        scratch_types=(
            [pltpu.VMEM((C,), jnp.int32)] * NBS
            + [pltpu.VMEM((C, L), jnp.float32)] * NBS
            + [pltpu.VMEM_SHARED((N, L), jnp.float32)]
            + [pltpu.SemaphoreType.DMA] * (3 * NBS)
        ),
    )
    def k(rows1_hbm, rows2_hbm, dst_hbm, out_hbm, *rest):
        di = rest[:NBS]
        rows = rest[NBS:2 * NBS]
        acc_sh = rest[2 * NBS]
        isem = rest[2 * NBS + 1:2 * NBS + 1 + NBS]
        rsem = rest[2 * NBS + 1 + NBS:2 * NBS + 1 + 2 * NBS]
        ssem = rest[2 * NBS + 1 + 2 * NBS:]
        cid = lax.axis_index("c")
        sid = lax.axis_index("s")
        base = sid * TPT

        def zrow(i, carry):
            def zcol(j, carry2):
                rows[0][i, pl.ds(j * 16, 16)] = jnp.zeros((16,), jnp.float32)
                return carry2
            return lax.fori_loop(0, L // 16, zcol, carry)
        lax.fori_loop(0, C, zrow, 0)

        for kk in range(-(-NDC // NS)):
            ch = sid + NS * kk

            @pl.when(ch < NDC)
            def _():
                pltpu.sync_copy(rows[0], acc_sh.at[pl.ds(ch * C, C)])

        plsc.subcore_barrier()

        def run(rows_hbm, eo):
            def grp(g, carry):
                for b in range(NBS):
                    c = g * NBS + b
                    off = base + c * C

                    @pl.when(g > 0)
                    def _():
                        pltpu.make_async_copy(rows[b], acc_sh.at[di[b]],
                                              ssem[b]).wait()

                    pltpu.async_copy(dst_hbm.at[pl.ds(eo + off, C)], di[b],
                                     isem[b])
                    pltpu.async_copy(rows_hbm.at[pl.ds(off, C)], rows[b],
                                     rsem[b])

                    cd = c - DS
                    bd = (b - DS) % NBS
                    offd = base + cd * C

                    @pl.when(cd >= 0)
                    def _():
                        pltpu.make_async_copy(dst_hbm.at[pl.ds(eo + offd, C)],
                                              di[bd], isem[bd]).wait()
                        pltpu.make_async_copy(rows_hbm.at[pl.ds(offd, C)],
                                              rows[bd], rsem[bd]).wait()
                        pltpu.async_copy(rows[bd], acc_sh.at[di[bd]], ssem[bd],
                                         add=True)
                return carry

            lax.fori_loop(0, NCHUNK // NBS, grp, 0)

            for bd in range(NBS - DS, NBS):
                offd = base + (NCHUNK - NBS + bd) * C
                pltpu.make_async_copy(dst_hbm.at[pl.ds(eo + offd, C)], di[bd],
                                      isem[bd]).wait()
                pltpu.make_async_copy(rows_hbm.at[pl.ds(offd, C)], rows[bd],
                                      rsem[bd]).wait()
                pltpu.async_copy(rows[bd], acc_sh.at[di[bd]], ssem[bd],
                                 add=True)
            for b in range(NBS):
                pltpu.make_async_copy(rows[b], acc_sh.at[di[b]],
                                      ssem[b]).wait()

        @pl.when(cid == 0)
        def _():
            run(rows1_hbm, 0)

        @pl.when(cid == 1)
        def _():
            run(rows2_hbm, H)

        plsc.subcore_barrier()

        for kk in range(-(-NDC // NS)):
            ch = sid + NS * kk

            @pl.when(ch < NDC)
            def _():
                pltpu.sync_copy(acc_sh.at[pl.ds(ch * C, C)], rows[0])
                pltpu.sync_copy(rows[0], out_hbm.at[cid, pl.ds(ch * C, C)])

    return k(rows1, rows2, dst)


def _vec(b):
    return b.reshape(1, -1)


def kernel(node_x, edge_attr, params, edge_index, edge_type):
    del edge_type
    src = edge_index[0]
    dst = edge_index[1]

    (en_w0, en_b0), (en_w1, en_b1) = params["enc_node"]["mlp"]
    en_g, en_bl = params["enc_node"]["ln"]
    (ee_w0, ee_b0), (ee_w1, ee_b1) = params["enc_edge"]["mlp"]
    ee_g, ee_bl = params["enc_edge"]["ln"]

    steps = []
    for st in params["proc"]:
        (mw0, mb0), (mw1, mb1) = st["msg"]["mlp"]
        mg, mbl = st["msg"]["ln"]
        (uw0, ub0), (uw1, ub1) = st["upd"]["mlp"]
        ug, ubl = st["upd"]["ln"]
        steps.append(dict(
            wa=mw0[:L], wb=mw0[L:2 * L], wc=mw0[2 * L:],
            mb0=_vec(mb0), mw1=mw1, mb1=_vec(mb1), mg=_vec(mg), mbl=_vec(mbl),
            wx=uw0[:L], wg=uw0[L:],
            ub0=_vec(ub0), uw1=uw1, ub1=_vec(ub1), ug=_vec(ug), ubl=_vec(ubl),
        ))
    (dw0, db0), (dw1, db1) = params["dec"]

    s0, s1 = steps
    x, p, q = _enc_node_pq(node_x, en_w0, _vec(en_b0), en_w1, _vec(en_b1),
                           _vec(en_g), _vec(en_bl), s0["wa"], s0["wb"])
    a1, b1 = _sc_gather2(p, q, dst, src, 0)
    a2, b2 = _sc_gather2(p, q, dst, src, H)
    enc_e = (ee_w0, _vec(ee_b0), ee_w1, _vec(ee_b1), _vec(ee_g), _vec(ee_bl))
    e1 = _msg0_update(a1, b1, edge_attr, 0,
                      *enc_e, s0["wc"], s0["mb0"], s0["mw1"], s0["mb1"],
                      s0["mg"], s0["mbl"])
    e2 = _msg0_update(a2, b2, edge_attr, H // BE,
                      *enc_e, s0["wc"], s0["mb0"], s0["mw1"], s0["mb1"],
                      s0["mg"], s0["mbl"])
    aggp = _sc_scatter(e1, e2, dst)
    x, p, q = _upd_pq(x, aggp, s0["wx"], s0["wg"], s0["ub0"], s0["uw1"],
                      s0["ub1"], s0["ug"], s0["ubl"], s1["wa"], s1["wb"])

    a1, b1 = _sc_gather2(p, q, dst, src, 0)
    a2, b2 = _sc_gather2(p, q, dst, src, H)
    e1 = _msg_update(a1, b1, e1, s1["wc"],
                     s1["mb0"], s1["mw1"], s1["mb1"], s1["mg"], s1["mbl"])
    e2 = _msg_update(a2, b2, e2, s1["wc"],
                     s1["mb0"], s1["mw1"], s1["mb1"], s1["mg"], s1["mbl"])
    aggp = _sc_scatter(e1, e2, dst)
    y = _upd_dec(x, aggp, s1["wx"], s1["wg"], s1["ub0"], s1["uw1"],
                 s1["ub1"], s1["ug"], s1["ubl"],
                 dw0, _vec(db0), dw1, _vec(db1))
    return y


# speedup vs baseline: 1.0343x; 1.0202x over previous
import functools

import jax
import jax.numpy as jnp
from jax import lax
from jax.experimental import pallas as pl
from jax.experimental.pallas import tpu as pltpu
from jax.experimental.pallas import tpu_sc as plsc

N = 10000
E = 320000
D_EDGE = 16
L = 128

BN = 2000
BE = 8000

NC = 2
NS = 16
NW = NC * NS
TPE = E // NW
H = E // 2
C = 40
TPT = H // NS
NCHUNK = TPT // C
NDC = N // C


def _ln(y, g, b):
    m = jnp.mean(y, axis=-1, keepdims=True)
    v = jnp.mean((y - m) ** 2, axis=-1, keepdims=True)
    return (y - m) / jnp.sqrt(v + 1e-5) * g + b


def _enc_node_body(x_ref, w0, b0, w1, b1, g, bl, wa, wb, xo, po, qo):
    h = jnp.maximum(x_ref[...] @ w0[...] + b0[...], 0.0)
    xn = _ln(h @ w1[...] + b1[...], g[...], bl[...])
    xo[...] = xn
    po[...] = xn @ wa[...]
    qo[...] = xn @ wb[...]


def _enc_node_pq(node_x, w0, b0, w1, b1, g, bl, wa, wb):
    full = pl.BlockSpec((L, L), lambda i: (0, 0))
    vec = pl.BlockSpec((1, L), lambda i: (0, 0))
    blk = pl.BlockSpec((BN, L), lambda i: (i, 0))
    return pl.pallas_call(
        _enc_node_body,
        grid=(N // BN,),
        in_specs=[blk, full, vec, full, vec, vec, vec, full, full],
        out_specs=[blk, blk, blk],
        out_shape=[jax.ShapeDtypeStruct((N, L), jnp.float32)] * 3,
    )(node_x, w0, b0, w1, b1, g, bl, wa, wb)


def _enc_edge_body(a_ref, w0, b0, w1, b1, g, bl, eo):
    h = jnp.maximum(a_ref[...] @ w0[...] + b0[...], 0.0)
    eo[...] = _ln(h @ w1[...] + b1[...], g[...], bl[...])


def _enc_edge(edge_attr, off, w0, b0, w1, b1, g, bl):
    vec = pl.BlockSpec((1, L), lambda i: (0, 0))
    return pl.pallas_call(
        _enc_edge_body,
        grid=(H // BE,),
        in_specs=[pl.BlockSpec((BE, D_EDGE), lambda i: (i + off, 0)),
                  pl.BlockSpec((D_EDGE, L), lambda i: (0, 0)),
                  vec,
                  pl.BlockSpec((L, L), lambda i: (0, 0)),
                  vec, vec, vec],
        out_specs=pl.BlockSpec((BE, L), lambda i: (i, 0)),
        out_shape=jax.ShapeDtypeStruct((H, L), jnp.float32),
    )(edge_attr, w0, b0, w1, b1, g, bl)


def _msg0_body(a_ref, b_ref, ea_ref, ew0, eb0, ew1, eb1, eg, ebl,
               wc, b0, w1, b1, g, bl, eo):
    he = jnp.maximum(ea_ref[...] @ ew0[...] + eb0[...], 0.0)
    e = _ln(he @ ew1[...] + eb1[...], eg[...], ebl[...])
    gsum = a_ref[...].astype(jnp.float32) + b_ref[...].astype(jnp.float32)
    pre = gsum + e @ wc[...] + b0[...]
    h = jnp.maximum(pre, 0.0)
    msg = _ln(h @ w1[...] + b1[...], g[...], bl[...])
    eo[...] = e + msg


def _msg0_update(a, b, edge_attr, off, ew0, eb0, ew1, eb1, eg, ebl,
                 wc, b0, w1, b1, g, bl):
    full = pl.BlockSpec((L, L), lambda i: (0, 0))
    vec = pl.BlockSpec((1, L), lambda i: (0, 0))
    blk = pl.BlockSpec((BE, L), lambda i: (i, 0))
    return pl.pallas_call(
        _msg0_body,
        grid=(H // BE,),
        in_specs=[blk, blk,
                  pl.BlockSpec((BE, D_EDGE), lambda i: (i + off, 0)),
                  pl.BlockSpec((D_EDGE, L), lambda i: (0, 0)),
                  vec, full, vec, vec, vec,
                  full, vec, full, vec, vec, vec],
        out_specs=pl.BlockSpec((BE, L), lambda i: (i, 0)),
        out_shape=jax.ShapeDtypeStruct((H, L), jnp.float32),
    )(a, b, edge_attr, ew0, eb0, ew1, eb1, eg, ebl,
      wc, b0, w1, b1, g, bl)


def _msg_body(a_ref, b_ref, e_ref, wc, b0, w1, b1, g, bl, eo):
    gsum = a_ref[...].astype(jnp.float32) + b_ref[...].astype(jnp.float32)
    pre = gsum + e_ref[...] @ wc[...] + b0[...]
    h = jnp.maximum(pre, 0.0)
    msg = _ln(h @ w1[...] + b1[...], g[...], bl[...])
    eo[...] = e_ref[...] + msg


def _msg_update(a, b, e, wc, b0, w1, b1, g, bl):
    full = pl.BlockSpec((L, L), lambda i: (0, 0))
    vec = pl.BlockSpec((1, L), lambda i: (0, 0))
    blk = pl.BlockSpec((BE, L), lambda i: (i, 0))
    return pl.pallas_call(
        _msg_body,
        grid=(H // BE,),
        in_specs=[blk, blk, blk, full, vec, full, vec, vec, vec],
        out_specs=blk,
        out_shape=jax.ShapeDtypeStruct((H, L), jnp.float32),
    )(a, b, e, wc, b0, w1, b1, g, bl)


def _upd_pq_body(x_ref, agg_ref, wx, wg, b0, w1, b1, g, bl, wa, wb,
                 xo, po, qo):
    agg = agg_ref[0] + agg_ref[1]
    pre = x_ref[...] @ wx[...] + agg @ wg[...] + b0[...]
    h = jnp.maximum(pre, 0.0)
    upd = _ln(h @ w1[...] + b1[...], g[...], bl[...])
    xn = x_ref[...] + upd
    xo[...] = xn
    po[...] = xn @ wa[...]
    qo[...] = xn @ wb[...]


def _upd_pq(x, aggp, wx, wg, b0, w1, b1, g, bl, wa, wb):
    full = pl.BlockSpec((L, L), lambda i: (0, 0))
    vec = pl.BlockSpec((1, L), lambda i: (0, 0))
    blk = pl.BlockSpec((BN, L), lambda i: (i, 0))
    ablk = pl.BlockSpec((2, BN, L), lambda i: (0, i, 0))
    return pl.pallas_call(
        _upd_pq_body,
        grid=(N // BN,),
        in_specs=[blk, ablk, full, full, vec, full, vec, vec, vec, full, full],
        out_specs=[blk, blk, blk],
        out_shape=[jax.ShapeDtypeStruct((N, L), jnp.float32)] * 3,
    )(x, aggp, wx, wg, b0, w1, b1, g, bl, wa, wb)


def _upd_dec_body(x_ref, agg_ref, wx, wg, b0, w1, b1, g, bl,
                  wd0, bd0, wd1, bd1, yo):
    agg = agg_ref[0] + agg_ref[1]
    pre = x_ref[...] @ wx[...] + agg @ wg[...] + b0[...]
    h = jnp.maximum(pre, 0.0)
    upd = _ln(h @ w1[...] + b1[...], g[...], bl[...])
    xn = x_ref[...] + upd
    hd = jnp.maximum(xn @ wd0[...] + bd0[...], 0.0)
    yo[...] = hd @ wd1[...] + bd1[...]


def _upd_dec(x, aggp, wx, wg, b0, w1, b1, g, bl, wd0, bd0, wd1, bd1):
    full = pl.BlockSpec((L, L), lambda i: (0, 0))
    vec = pl.BlockSpec((1, L), lambda i: (0, 0))
    blk = pl.BlockSpec((BN, L), lambda i: (i, 0))
    ablk = pl.BlockSpec((2, BN, L), lambda i: (0, i, 0))
    return pl.pallas_call(
        _upd_dec_body,
        grid=(N // BN,),
        in_specs=[blk, ablk, full, full, vec, full, vec, vec, vec,
                  full, vec,
                  pl.BlockSpec((L, 3), lambda i: (0, 0)),
                  pl.BlockSpec((1, 3), lambda i: (0, 0))],
        out_specs=pl.BlockSpec((BN, 3), lambda i: (i, 0)),
        out_shape=jax.ShapeDtypeStruct((N, 3), jnp.float32),
    )(x, aggp, wx, wg, b0, w1, b1, g, bl, wd0, bd0, wd1, bd1)


NBG = 5
DG = 1
CG = C
NCH_G = TPT // CG
NGRP_G = NCH_G // NBG
NSTAGE = N // CG


def _sc_gather2(p, q, dst, src, eo):
    mesh = plsc.VectorSubcoreMesh(core_axis_name="c", subcore_axis_name="s")

    @functools.partial(
        pl.kernel, mesh=mesh,
        out_type=[jax.ShapeDtypeStruct((H, L), jnp.float32)] * 2,
        scratch_types=(
            [pltpu.VMEM((TPT,), jnp.int32)]
            + [pltpu.VMEM((CG, L), jnp.float32)] * NBG
            + [pltpu.VMEM_SHARED((N, L), jnp.float32)]
            + [pltpu.SemaphoreType.DMA] * (2 * NBG)
        ),
    )
    def k(p_hbm, q_hbm, dst_hbm, src_hbm, a_hbm, b_hbm, idxall, *rest):
        rows = rest[:NBG]
        tbl_sh = rest[NBG]
        gsem = rest[NBG + 1:NBG + 1 + NBG]
        wsem = rest[NBG + 1 + NBG:]
        cid = lax.axis_index("c")
        sid = lax.axis_index("s")
        base = sid * TPT

        def run(tbl_hbm, idx_hbm, out_hbm):
            for st in range(-(-NSTAGE // NS)):
                ch = sid + NS * st

                @pl.when(ch < NSTAGE)
                def _():
                    pltpu.sync_copy(tbl_hbm.at[pl.ds(ch * CG, CG)], rows[0])
                    pltpu.sync_copy(rows[0], tbl_sh.at[pl.ds(ch * CG, CG)])

            pltpu.sync_copy(idx_hbm.at[pl.ds(eo + base, TPT)], idxall)
            plsc.subcore_barrier()

            def grp(g, carry):
                for b in range(NBG):
                    c = g * NBG + b

                    @pl.when(g > 0)
                    def _():
                        pltpu.make_async_copy(
                            rows[b], out_hbm.at[pl.ds(base + b * CG, CG)],
                            wsem[b]).wait()

                    pltpu.async_copy(
                        tbl_sh.at[idxall.at[pl.ds(c * CG, CG)]], rows[b],
                        gsem[b])

                    cd = c - DG
                    bd = (b - DG) % NBG

                    @pl.when(cd >= 0)
                    def _():
                        pltpu.make_async_copy(
                            tbl_sh.at[idxall.at[pl.ds(bd * CG, CG)]], rows[bd],
                            gsem[bd]).wait()
                        pltpu.async_copy(
                            rows[bd], out_hbm.at[pl.ds(base + cd * CG, CG)],
                            wsem[bd])
                return carry

            lax.fori_loop(0, NGRP_G, grp, 0)

            for bd in range(NBG - DG, NBG):
                cd = NCH_G - NBG + bd
                pltpu.make_async_copy(
                    tbl_sh.at[idxall.at[pl.ds(bd * CG, CG)]], rows[bd],
                    gsem[bd]).wait()
                pltpu.async_copy(
                    rows[bd], out_hbm.at[pl.ds(base + cd * CG, CG)], wsem[bd])
            for b in range(NBG):
                pltpu.make_async_copy(
                    rows[b], out_hbm.at[pl.ds(base + b * CG, CG)],
                    wsem[b]).wait()

        @pl.when(cid == 0)
        def _():
            run(p_hbm, dst_hbm, a_hbm)

        @pl.when(cid == 1)
        def _():
            run(q_hbm, src_hbm, b_hbm)

    return k(p, q, dst, src)


NBS = 5
DS = 3


def _sc_scatter(rows1, rows2, dst):
    mesh = plsc.VectorSubcoreMesh(core_axis_name="c", subcore_axis_name="s")

    @functools.partial(
        pl.kernel, mesh=mesh,
        out_type=jax.ShapeDtypeStruct((NC, N, L), jnp.float32),
        scratch_types=(
            [pltpu.VMEM((C,), jnp.int32)] * NBS
            + [pltpu.VMEM((C, L), jnp.float32)] * NBS
            + [pltpu.VMEM_SHARED((N, L), jnp.float32)]
            + [pltpu.SemaphoreType.DMA] * (3 * NBS)
        ),
    )
    def k(rows1_hbm, rows2_hbm, dst_hbm, out_hbm, *rest):
        di = rest[:NBS]
        rows = rest[NBS:2 * NBS]
        acc_sh = rest[2 * NBS]
        isem = rest[2 * NBS + 1:2 * NBS + 1 + NBS]
        rsem = rest[2 * NBS + 1 + NBS:2 * NBS + 1 + 2 * NBS]
        ssem = rest[2 * NBS + 1 + 2 * NBS:]
        cid = lax.axis_index("c")
        sid = lax.axis_index("s")
        base = sid * TPT

        def zrow(i, carry):
            def zcol(j, carry2):
                rows[0][i, pl.ds(j * 16, 16)] = jnp.zeros((16,), jnp.float32)
                return carry2
            return lax.fori_loop(0, L // 16, zcol, carry)
        lax.fori_loop(0, C, zrow, 0)

        for kk in range(-(-NDC // NS)):
            ch = sid + NS * kk

            @pl.when(ch < NDC)
            def _():
                pltpu.sync_copy(rows[0], acc_sh.at[pl.ds(ch * C, C)])

        plsc.subcore_barrier()

        def run(rows_hbm, eo):
            def grp(g, carry):
                for b in range(NBS):
                    c = g * NBS + b
                    off = base + c * C

                    @pl.when(g > 0)
                    def _():
                        pltpu.make_async_copy(rows[b], acc_sh.at[di[b]],
                                              ssem[b]).wait()

                    pltpu.async_copy(dst_hbm.at[pl.ds(eo + off, C)], di[b],
                                     isem[b])
                    pltpu.async_copy(rows_hbm.at[pl.ds(off, C)], rows[b],
                                     rsem[b])

                    cd = c - DS
                    bd = (b - DS) % NBS
                    offd = base + cd * C

                    @pl.when(cd >= 0)
                    def _():
                        pltpu.make_async_copy(dst_hbm.at[pl.ds(eo + offd, C)],
                                              di[bd], isem[bd]).wait()
                        pltpu.make_async_copy(rows_hbm.at[pl.ds(offd, C)],
                                              rows[bd], rsem[bd]).wait()
                        pltpu.async_copy(rows[bd], acc_sh.at[di[bd]], ssem[bd],
                                         add=True)
                return carry

            lax.fori_loop(0, NCHUNK // NBS, grp, 0)

            for bd in range(NBS - DS, NBS):
                offd = base + (NCHUNK - NBS + bd) * C
                pltpu.make_async_copy(dst_hbm.at[pl.ds(eo + offd, C)], di[bd],
                                      isem[bd]).wait()
                pltpu.make_async_copy(rows_hbm.at[pl.ds(offd, C)], rows[bd],
                                      rsem[bd]).wait()
                pltpu.async_copy(rows[bd], acc_sh.at[di[bd]], ssem[bd],
                                 add=True)
            for b in range(NBS):
                pltpu.make_async_copy(rows[b], acc_sh.at[di[b]],
                                      ssem[b]).wait()

        @pl.when(cid == 0)
        def _():
            run(rows1_hbm, 0)

        @pl.when(cid == 1)
        def _():
            run(rows2_hbm, H)

        plsc.subcore_barrier()

        for kk in range(-(-NDC // NS)):
            ch = sid + NS * kk

            @pl.when(ch < NDC)
            def _():
                pltpu.sync_copy(acc_sh.at[pl.ds(ch * C, C)], rows[0])
                pltpu.sync_copy(rows[0], out_hbm.at[cid, pl.ds(ch * C, C)])

    return k(rows1, rows2, dst)


def _vec(b):
    return b.reshape(1, -1)


def kernel(node_x, edge_attr, params, edge_index, edge_type):
    del edge_type
    src = edge_index[0]
    dst = edge_index[1]

    (en_w0, en_b0), (en_w1, en_b1) = params["enc_node"]["mlp"]
    en_g, en_bl = params["enc_node"]["ln"]
    (ee_w0, ee_b0), (ee_w1, ee_b1) = params["enc_edge"]["mlp"]
    ee_g, ee_bl = params["enc_edge"]["ln"]

    steps = []
    for st in params["proc"]:
        (mw0, mb0), (mw1, mb1) = st["msg"]["mlp"]
        mg, mbl = st["msg"]["ln"]
        (uw0, ub0), (uw1, ub1) = st["upd"]["mlp"]
        ug, ubl = st["upd"]["ln"]
        steps.append(dict(
            wa=mw0[:L], wb=mw0[L:2 * L], wc=mw0[2 * L:],
            mb0=_vec(mb0), mw1=mw1, mb1=_vec(mb1), mg=_vec(mg), mbl=_vec(mbl),
            wx=uw0[:L], wg=uw0[L:],
            ub0=_vec(ub0), uw1=uw1, ub1=_vec(ub1), ug=_vec(ug), ubl=_vec(ubl),
        ))
    (dw0, db0), (dw1, db1) = params["dec"]

    s0, s1 = steps
    x, p, q = _enc_node_pq(node_x, en_w0, _vec(en_b0), en_w1, _vec(en_b1),
                           _vec(en_g), _vec(en_bl), s0["wa"], s0["wb"])
    a1, b1 = _sc_gather2(p, q, dst, src, 0)
    a2, b2 = _sc_gather2(p, q, dst, src, H)
    enc_e = (ee_w0, _vec(ee_b0), ee_w1, _vec(ee_b1), _vec(ee_g), _vec(ee_bl))
    e1 = _msg0_update(a1, b1, edge_attr, 0,
                      *enc_e, s0["wc"], s0["mb0"], s0["mw1"], s0["mb1"],
                      s0["mg"], s0["mbl"])
    e2 = _msg0_update(a2, b2, edge_attr, H // BE,
                      *enc_e, s0["wc"], s0["mb0"], s0["mw1"], s0["mb1"],
                      s0["mg"], s0["mbl"])
    aggp = _sc_scatter(e1, e2, dst)
    x, p, q = _upd_pq(x, aggp, s0["wx"], s0["wg"], s0["ub0"], s0["uw1"],
                      s0["ub1"], s0["ug"], s0["ubl"], s1["wa"], s1["wb"])

    a1, b1 = _sc_gather2(p, q, dst, src, 0)
    a2, b2 = _sc_gather2(p, q, dst, src, H)
    e1 = _msg_update(a1, b1, e1, s1["wc"],
                     s1["mb0"], s1["mw1"], s1["mb1"], s1["mg"], s1["mbl"])
    e2 = _msg_update(a2, b2, e2, s1["wc"],
                     s1["mb0"], s1["mw1"], s1["mb1"], s1["mg"], s1["mbl"])
    aggp = _sc_scatter(e1, e2, dst)
    y = _upd_dec(x, aggp, s1["wx"], s1["wg"], s1["ub0"], s1["uw1"],
                 s1["ub1"], s1["ug"], s1["ubl"],
                 dw0, _vec(db0), dw1, _vec(db1))
    return y


# speedup vs baseline: 1.0375x; 1.0031x over previous
import functools

import jax
import jax.numpy as jnp
from jax import lax
from jax.experimental import pallas as pl
from jax.experimental.pallas import tpu as pltpu
from jax.experimental.pallas import tpu_sc as plsc

N = 10000
E = 320000
D_EDGE = 16
L = 128

BN = 2000
BE = 8000

NC = 2
NS = 16
NW = NC * NS
TPE = E // NW
H = E // 2
C = 40
TPT = H // NS
NCHUNK = TPT // C
NDC = N // C


def _ln(y, g, b):
    m = jnp.mean(y, axis=-1, keepdims=True)
    v = jnp.mean((y - m) ** 2, axis=-1, keepdims=True)
    return (y - m) / jnp.sqrt(v + 1e-5) * g + b


def _enc_node_body(x_ref, w0, b0, w1, b1, g, bl, wa, wb, xo, po, qo):
    h = jnp.maximum(x_ref[...] @ w0[...] + b0[...], 0.0)
    xn = _ln(h @ w1[...] + b1[...], g[...], bl[...])
    xo[...] = xn
    po[...] = xn @ wa[...]
    qo[...] = xn @ wb[...]


def _enc_node_pq(node_x, w0, b0, w1, b1, g, bl, wa, wb):
    full = pl.BlockSpec((L, L), lambda i: (0, 0))
    vec = pl.BlockSpec((1, L), lambda i: (0, 0))
    blk = pl.BlockSpec((BN, L), lambda i: (i, 0))
    return pl.pallas_call(
        _enc_node_body,
        grid=(N // BN,),
        in_specs=[blk, full, vec, full, vec, vec, vec, full, full],
        out_specs=[blk, blk, blk],
        out_shape=[jax.ShapeDtypeStruct((N, L), jnp.float32)] * 3,
    )(node_x, w0, b0, w1, b1, g, bl, wa, wb)


def _msg0_body(a_ref, b_ref, ea_ref, ew0, eb0, ew1, eb1, eg, ebl,
               wc, b0, w1, b1, g, bl, eo):
    he = jnp.maximum(ea_ref[...] @ ew0[...] + eb0[...], 0.0)
    e = _ln(he @ ew1[...] + eb1[...], eg[...], ebl[...])
    gsum = a_ref[...].astype(jnp.float32) + b_ref[...].astype(jnp.float32)
    pre = gsum + e @ wc[...] + b0[...]
    h = jnp.maximum(pre, 0.0)
    msg = _ln(h @ w1[...] + b1[...], g[...], bl[...])
    eo[...] = e + msg


def _msg0_update(a, b, edge_attr, off, ew0, eb0, ew1, eb1, eg, ebl,
                 wc, b0, w1, b1, g, bl):
    full = pl.BlockSpec((L, L), lambda i: (0, 0))
    vec = pl.BlockSpec((1, L), lambda i: (0, 0))
    blk = pl.BlockSpec((BE, L), lambda i: (i, 0))
    return pl.pallas_call(
        _msg0_body,
        grid=(H // BE,),
        in_specs=[blk, blk,
                  pl.BlockSpec((BE, D_EDGE), lambda i: (i + off, 0)),
                  pl.BlockSpec((D_EDGE, L), lambda i: (0, 0)),
                  vec, full, vec, vec, vec,
                  full, vec, full, vec, vec, vec],
        out_specs=pl.BlockSpec((BE, L), lambda i: (i, 0)),
        out_shape=jax.ShapeDtypeStruct((H, L), jnp.float32),
    )(a, b, edge_attr, ew0, eb0, ew1, eb1, eg, ebl,
      wc, b0, w1, b1, g, bl)


def _msg_body(a_ref, b_ref, e_ref, wc, b0, w1, b1, g, bl, eo):
    gsum = a_ref[...].astype(jnp.float32) + b_ref[...].astype(jnp.float32)
    pre = gsum + e_ref[...] @ wc[...] + b0[...]
    h = jnp.maximum(pre, 0.0)
    msg = _ln(h @ w1[...] + b1[...], g[...], bl[...])
    eo[...] = e_ref[...] + msg


def _msg_update(a, b, e, wc, b0, w1, b1, g, bl):
    full = pl.BlockSpec((L, L), lambda i: (0, 0))
    vec = pl.BlockSpec((1, L), lambda i: (0, 0))
    blk = pl.BlockSpec((BE, L), lambda i: (i, 0))
    return pl.pallas_call(
        _msg_body,
        grid=(H // BE,),
        in_specs=[blk, blk, blk, full, vec, full, vec, vec, vec],
        out_specs=blk,
        out_shape=jax.ShapeDtypeStruct((H, L), jnp.float32),
    )(a, b, e, wc, b0, w1, b1, g, bl)


def _upd_pq_body(x_ref, agg_ref, wx, wg, b0, w1, b1, g, bl, wa, wb,
                 xo, po, qo):
    agg = agg_ref[0] + agg_ref[1]
    pre = x_ref[...] @ wx[...] + agg @ wg[...] + b0[...]
    h = jnp.maximum(pre, 0.0)
    upd = _ln(h @ w1[...] + b1[...], g[...], bl[...])
    xn = x_ref[...] + upd
    xo[...] = xn
    po[...] = xn @ wa[...]
    qo[...] = xn @ wb[...]


def _upd_pq(x, aggp, wx, wg, b0, w1, b1, g, bl, wa, wb):
    full = pl.BlockSpec((L, L), lambda i: (0, 0))
    vec = pl.BlockSpec((1, L), lambda i: (0, 0))
    blk = pl.BlockSpec((BN, L), lambda i: (i, 0))
    ablk = pl.BlockSpec((2, BN, L), lambda i: (0, i, 0))
    return pl.pallas_call(
        _upd_pq_body,
        grid=(N // BN,),
        in_specs=[blk, ablk, full, full, vec, full, vec, vec, vec, full, full],
        out_specs=[blk, blk, blk],
        out_shape=[jax.ShapeDtypeStruct((N, L), jnp.float32)] * 3,
    )(x, aggp, wx, wg, b0, w1, b1, g, bl, wa, wb)


def _upd_dec_body(x_ref, agg_ref, wx, wg, b0, w1, b1, g, bl,
                  wd0, bd0, wd1, bd1, yo):
    agg = agg_ref[0] + agg_ref[1]
    pre = x_ref[...] @ wx[...] + agg @ wg[...] + b0[...]
    h = jnp.maximum(pre, 0.0)
    upd = _ln(h @ w1[...] + b1[...], g[...], bl[...])
    xn = x_ref[...] + upd
    hd = jnp.maximum(xn @ wd0[...] + bd0[...], 0.0)
    yo[...] = hd @ wd1[...] + bd1[...]


def _upd_dec(x, aggp, wx, wg, b0, w1, b1, g, bl, wd0, bd0, wd1, bd1):
    full = pl.BlockSpec((L, L), lambda i: (0, 0))
    vec = pl.BlockSpec((1, L), lambda i: (0, 0))
    blk = pl.BlockSpec((BN, L), lambda i: (i, 0))
    ablk = pl.BlockSpec((2, BN, L), lambda i: (0, i, 0))
    return pl.pallas_call(
        _upd_dec_body,
        grid=(N // BN,),
        in_specs=[blk, ablk, full, full, vec, full, vec, vec, vec,
                  full, vec,
                  pl.BlockSpec((L, 3), lambda i: (0, 0)),
                  pl.BlockSpec((1, 3), lambda i: (0, 0))],
        out_specs=pl.BlockSpec((BN, 3), lambda i: (i, 0)),
        out_shape=jax.ShapeDtypeStruct((N, 3), jnp.float32),
    )(x, aggp, wx, wg, b0, w1, b1, g, bl, wd0, bd0, wd1, bd1)


NBG = 5
DG = 1
CG = C
NCH_G = TPT // CG
NGRP_G = NCH_G // NBG
NSTAGE = N // CG


def _sc_gather2(p, q, dst, src, eo):
    mesh = plsc.VectorSubcoreMesh(core_axis_name="c", subcore_axis_name="s")

    @functools.partial(
        pl.kernel, mesh=mesh,
        out_type=[jax.ShapeDtypeStruct((H, L), jnp.float32)] * 2,
        scratch_types=(
            [pltpu.VMEM((TPT,), jnp.int32)]
            + [pltpu.VMEM((CG, L), jnp.float32)] * NBG
            + [pltpu.VMEM_SHARED((N, L), jnp.float32)]
            + [pltpu.SemaphoreType.DMA] * (2 * NBG)
        ),
    )
    def k(p_hbm, q_hbm, dst_hbm, src_hbm, a_hbm, b_hbm, idxall, *rest):
        rows = rest[:NBG]
        tbl_sh = rest[NBG]
        gsem = rest[NBG + 1:NBG + 1 + NBG]
        wsem = rest[NBG + 1 + NBG:]
        cid = lax.axis_index("c")
        sid = lax.axis_index("s")
        base = sid * TPT

        def run(tbl_hbm, idx_hbm, out_hbm):
            for st in range(-(-NSTAGE // NS)):
                ch = sid + NS * st

                @pl.when(ch < NSTAGE)
                def _():
                    pltpu.sync_copy(tbl_hbm.at[pl.ds(ch * CG, CG)], rows[0])
                    pltpu.sync_copy(rows[0], tbl_sh.at[pl.ds(ch * CG, CG)])

            pltpu.sync_copy(idx_hbm.at[pl.ds(eo + base, TPT)], idxall)
            plsc.subcore_barrier()

            def grp(g, carry):
                for b in range(NBG):
                    c = g * NBG + b

                    @pl.when(g > 0)
                    def _():
                        pltpu.make_async_copy(
                            rows[b], out_hbm.at[pl.ds(base + b * CG, CG)],
                            wsem[b]).wait()

                    pltpu.async_copy(
                        tbl_sh.at[idxall.at[pl.ds(c * CG, CG)]], rows[b],
                        gsem[b])

                    cd = c - DG
                    bd = (b - DG) % NBG

                    @pl.when(cd >= 0)
                    def _():
                        pltpu.make_async_copy(
                            tbl_sh.at[idxall.at[pl.ds(bd * CG, CG)]], rows[bd],
                            gsem[bd]).wait()
                        pltpu.async_copy(
                            rows[bd], out_hbm.at[pl.ds(base + cd * CG, CG)],
                            wsem[bd])
                return carry

            lax.fori_loop(0, NGRP_G, grp, 0)

            for bd in range(NBG - DG, NBG):
                cd = NCH_G - NBG + bd
                pltpu.make_async_copy(
                    tbl_sh.at[idxall.at[pl.ds(bd * CG, CG)]], rows[bd],
                    gsem[bd]).wait()
                pltpu.async_copy(
                    rows[bd], out_hbm.at[pl.ds(base + cd * CG, CG)], wsem[bd])
            for b in range(NBG):
                pltpu.make_async_copy(
                    rows[b], out_hbm.at[pl.ds(base + b * CG, CG)],
                    wsem[b]).wait()

        @pl.when(cid == 0)
        def _():
            run(p_hbm, dst_hbm, a_hbm)

        @pl.when(cid == 1)
        def _():
            run(q_hbm, src_hbm, b_hbm)

    return k(p, q, dst, src)


NBS = 5
DS = 4


def _sc_scatter(rows1, rows2, dst):
    mesh = plsc.VectorSubcoreMesh(core_axis_name="c", subcore_axis_name="s")

    @functools.partial(
        pl.kernel, mesh=mesh,
        out_type=jax.ShapeDtypeStruct((NC, N, L), jnp.float32),
        scratch_types=(
            [pltpu.VMEM((C,), jnp.int32)] * NBS
            + [pltpu.VMEM((C, L), jnp.float32)] * NBS
            + [pltpu.VMEM_SHARED((N, L), jnp.float32)]
            + [pltpu.SemaphoreType.DMA] * (3 * NBS)
        ),
    )
    def k(rows1_hbm, rows2_hbm, dst_hbm, out_hbm, *rest):
        di = rest[:NBS]
        rows = rest[NBS:2 * NBS]
        acc_sh = rest[2 * NBS]
        isem = rest[2 * NBS + 1:2 * NBS + 1 + NBS]
        rsem = rest[2 * NBS + 1 + NBS:2 * NBS + 1 + 2 * NBS]
        ssem = rest[2 * NBS + 1 + 2 * NBS:]
        cid = lax.axis_index("c")
        sid = lax.axis_index("s")
        base = sid * TPT

        def zrow(i, carry):
            def zcol(j, carry2):
                rows[0][i, pl.ds(j * 16, 16)] = jnp.zeros((16,), jnp.float32)
                return carry2
            return lax.fori_loop(0, L // 16, zcol, carry)
        lax.fori_loop(0, C, zrow, 0)

        for kk in range(-(-NDC // NS)):
            ch = sid + NS * kk

            @pl.when(ch < NDC)
            def _():
                pltpu.sync_copy(rows[0], acc_sh.at[pl.ds(ch * C, C)])

        plsc.subcore_barrier()

        def run(rows_hbm, eo):
            def grp(g, carry):
                for b in range(NBS):
                    c = g * NBS + b
                    off = base + c * C

                    @pl.when(g > 0)
                    def _():
                        pltpu.make_async_copy(rows[b], acc_sh.at[di[b]],
                                              ssem[b]).wait()

                    pltpu.async_copy(dst_hbm.at[pl.ds(eo + off, C)], di[b],
                                     isem[b])
                    pltpu.async_copy(rows_hbm.at[pl.ds(off, C)], rows[b],
                                     rsem[b])

                    cd = c - DS
                    bd = (b - DS) % NBS
                    offd = base + cd * C

                    @pl.when(cd >= 0)
                    def _():
                        pltpu.make_async_copy(dst_hbm.at[pl.ds(eo + offd, C)],
                                              di[bd], isem[bd]).wait()
                        pltpu.make_async_copy(rows_hbm.at[pl.ds(offd, C)],
                                              rows[bd], rsem[bd]).wait()
                        pltpu.async_copy(rows[bd], acc_sh.at[di[bd]], ssem[bd],
                                         add=True)
                return carry

            lax.fori_loop(0, NCHUNK // NBS, grp, 0)

            for bd in range(NBS - DS, NBS):
                offd = base + (NCHUNK - NBS + bd) * C
                pltpu.make_async_copy(dst_hbm.at[pl.ds(eo + offd, C)], di[bd],
                                      isem[bd]).wait()
                pltpu.make_async_copy(rows_hbm.at[pl.ds(offd, C)], rows[bd],
                                      rsem[bd]).wait()
                pltpu.async_copy(rows[bd], acc_sh.at[di[bd]], ssem[bd],
                                 add=True)
            for b in range(NBS):
                pltpu.make_async_copy(rows[b], acc_sh.at[di[b]],
                                      ssem[b]).wait()

        @pl.when(cid == 0)
        def _():
            run(rows1_hbm, 0)

        @pl.when(cid == 1)
        def _():
            run(rows2_hbm, H)

        plsc.subcore_barrier()

        for kk in range(-(-NDC // NS)):
            ch = sid + NS * kk

            @pl.when(ch < NDC)
            def _():
                pltpu.sync_copy(acc_sh.at[pl.ds(ch * C, C)], rows[0])
                pltpu.sync_copy(rows[0], out_hbm.at[cid, pl.ds(ch * C, C)])

    return k(rows1, rows2, dst)


def _vec(b):
    return b.reshape(1, -1)


def kernel(node_x, edge_attr, params, edge_index, edge_type):
    del edge_type
    src = edge_index[0]
    dst = edge_index[1]

    (en_w0, en_b0), (en_w1, en_b1) = params["enc_node"]["mlp"]
    en_g, en_bl = params["enc_node"]["ln"]
    (ee_w0, ee_b0), (ee_w1, ee_b1) = params["enc_edge"]["mlp"]
    ee_g, ee_bl = params["enc_edge"]["ln"]

    steps = []
    for st in params["proc"]:
        (mw0, mb0), (mw1, mb1) = st["msg"]["mlp"]
        mg, mbl = st["msg"]["ln"]
        (uw0, ub0), (uw1, ub1) = st["upd"]["mlp"]
        ug, ubl = st["upd"]["ln"]
        steps.append(dict(
            wa=mw0[:L], wb=mw0[L:2 * L], wc=mw0[2 * L:],
            mb0=_vec(mb0), mw1=mw1, mb1=_vec(mb1), mg=_vec(mg), mbl=_vec(mbl),
            wx=uw0[:L], wg=uw0[L:],
            ub0=_vec(ub0), uw1=uw1, ub1=_vec(ub1), ug=_vec(ug), ubl=_vec(ubl),
        ))
    (dw0, db0), (dw1, db1) = params["dec"]

    s0, s1 = steps
    x, p, q = _enc_node_pq(node_x, en_w0, _vec(en_b0), en_w1, _vec(en_b1),
                           _vec(en_g), _vec(en_bl), s0["wa"], s0["wb"])
    a1, b1 = _sc_gather2(p, q, dst, src, 0)
    a2, b2 = _sc_gather2(p, q, dst, src, H)
    enc_e = (ee_w0, _vec(ee_b0), ee_w1, _vec(ee_b1), _vec(ee_g), _vec(ee_bl))
    e1 = _msg0_update(a1, b1, edge_attr, 0,
                      *enc_e, s0["wc"], s0["mb0"], s0["mw1"], s0["mb1"],
                      s0["mg"], s0["mbl"])
    e2 = _msg0_update(a2, b2, edge_attr, H // BE,
                      *enc_e, s0["wc"], s0["mb0"], s0["mw1"], s0["mb1"],
                      s0["mg"], s0["mbl"])
    aggp = _sc_scatter(e1, e2, dst)
    x, p, q = _upd_pq(x, aggp, s0["wx"], s0["wg"], s0["ub0"], s0["uw1"],
                      s0["ub1"], s0["ug"], s0["ubl"], s1["wa"], s1["wb"])

    a1, b1 = _sc_gather2(p, q, dst, src, 0)
    a2, b2 = _sc_gather2(p, q, dst, src, H)
    e1 = _msg_update(a1, b1, e1, s1["wc"],
                     s1["mb0"], s1["mw1"], s1["mb1"], s1["mg"], s1["mbl"])
    e2 = _msg_update(a2, b2, e2, s1["wc"],
                     s1["mb0"], s1["mw1"], s1["mb1"], s1["mg"], s1["mbl"])
    aggp = _sc_scatter(e1, e2, dst)
    y = _upd_dec(x, aggp, s1["wx"], s1["wg"], s1["ub0"], s1["uw1"],
                 s1["ub1"], s1["ug"], s1["ubl"],
                 dw0, _vec(db0), dw1, _vec(db1))
    return y
